# Initial kernel scaffold; baseline (speedup 1.0000x reference)
import jax
import jax.numpy as jnp
from jax.experimental import pallas as pl


def kernel(node_feature, node_type, edge_time, edge_index, edge_type, W_in, rel_emb, time_emb, W_out, b_out):
    raise NotImplementedError("write your pallas kernel here")


# speedup vs baseline: 4.4323x; 4.4323x over previous
import jax
import jax.numpy as jnp
from jax import lax
from jax.experimental import pallas as pl
from jax.experimental.pallas import tpu as pltpu
from jax.experimental.pallas import tpu_sc as plsc

N = 10000
E = 320000
D = 128
HID = 128
NTYPES = 3
NREL = 5
NTIME = 120

NC = 2
NS = 16
CHUNK = 64

EPAD = 320512
CHUNKS_PER_TILE = EPAD // (CHUNK * NS)
HSPLIT = 156
NPAD = 10112
ROWS_PER_TILE = NPAD // NS
HCOLS = 64
HWORDS = NPAD * HCOLS
HIST_PER_TILE = HWORDS // NS
TRASH = N * HCOLS


def _adapt_body(x_ref, t_ref, w_ref, h_ref):
    x = x_ref[...]
    h0 = jnp.dot(x, w_ref[0], preferred_element_type=jnp.float32)
    h1 = jnp.dot(x, w_ref[1], preferred_element_type=jnp.float32)
    h2 = jnp.dot(x, w_ref[2], preferred_element_type=jnp.float32)
    t = t_ref[...]
    h_ref[...] = jnp.where(t == 0, h0, jnp.where(t == 1, h1, h2))


def _adapt(node_feature, node_type, W_in):
    blk = 1000
    grid = N // blk
    return pl.pallas_call(
        _adapt_body,
        grid=(grid,),
        in_specs=[
            pl.BlockSpec((blk, D), lambda i: (i, 0)),
            pl.BlockSpec((blk, 1), lambda i: (i, 0)),
            pl.BlockSpec((NTYPES, D, HID), lambda i: (0, 0, 0)),
        ],
        out_specs=pl.BlockSpec((blk, HID), lambda i: (i, 0)),
        out_shape=jax.ShapeDtypeStruct((N, HID), jnp.float32),
    )(node_feature, node_type.reshape(N, 1), W_in)


def _sc_edge_body(h, srcp, dstp, ttp, etp, zrows, zflat,
                  agg_out, hist_out,
                  agg_sh, hist_sh,
                  src_v, dst_v, tt_v, et_v, f1_v, f2_v,
                  ones_v, msg_v, sem):
    c = lax.axis_index("c")
    s = lax.axis_index("s")

    pltpu.sync_copy(zrows, agg_sh.at[pl.ds(s * ROWS_PER_TILE, ROWS_PER_TILE)])
    pltpu.sync_copy(zflat, hist_sh.at[pl.ds(s * HIST_PER_TILE, HIST_PER_TILE)])
    for k in range(CHUNK // 16):
        ones_v[pl.ds(k * 16, 16)] = jnp.full((16,), 1.0, jnp.float32)
    plsc.subcore_barrier()

    lo = c * HCOLS

    def body(j, carry):
        base = pl.multiple_of((s * CHUNKS_PER_TILE + j) * CHUNK, CHUNK)
        pltpu.sync_copy(dstp.at[pl.ds(base, CHUNK)], dst_v)
        pltpu.sync_copy(ttp.at[pl.ds(base, CHUNK)], tt_v)
        pltpu.sync_copy(etp.at[pl.ds(base, CHUNK)], et_v)

        my_agg = jnp.where(c == 0, j < HSPLIT, j >= HSPLIT)

        @pl.when(my_agg)
        def _():
            pltpu.sync_copy(srcp.at[pl.ds(base, CHUNK)], src_v)
            pltpu.async_copy(h.at[src_v], msg_v, sem).wait()
            pltpu.sync_copy(msg_v, agg_sh.at[dst_v], add=True)

        for k in range(CHUNK // 16):
            sl = pl.ds(k * 16, 16)
            d = dst_v[sl]
            t = tt_v[sl]
            d64 = d * HCOLS
            trash = TRASH + ((d64 + t) & 4095)
            own = (t >= lo) & (t < lo + HCOLS)
            f1_v[sl] = jnp.where(own, d64 + (t - lo), trash)
            f2_v[sl] = d64 + ((NTIME - HCOLS) + et_v[sl])
        pltpu.sync_copy(ones_v, hist_sh.at[f1_v], add=True)

        @pl.when(c == 1)
        def _():
            pltpu.sync_copy(ones_v, hist_sh.at[f2_v], add=True)
        return carry

    lax.fori_loop(0, CHUNKS_PER_TILE, body, 0)
    plsc.subcore_barrier()

    pltpu.sync_copy(agg_sh.at[pl.ds(s * ROWS_PER_TILE, ROWS_PER_TILE)],
                    agg_out.at[c, pl.ds(s * ROWS_PER_TILE, ROWS_PER_TILE)])
    pltpu.sync_copy(hist_sh.at[pl.ds(s * HIST_PER_TILE, HIST_PER_TILE)],
                    hist_out.at[c, pl.ds(s * HIST_PER_TILE, HIST_PER_TILE)])


def _sc_edge(h, srcp, dstp, ttp, etp, zrows, zflat):
    mesh = plsc.VectorSubcoreMesh(core_axis_name="c", subcore_axis_name="s",
                                  num_cores=NC, num_subcores=NS)
    kern = pl.kernel(
        _sc_edge_body,
        out_type=[
            jax.ShapeDtypeStruct((NC, NPAD, HID), jnp.float32),
            jax.ShapeDtypeStruct((NC, HWORDS), jnp.float32),
        ],
        mesh=mesh,
        scratch_types=[
            pltpu.VMEM_SHARED((NPAD, HID), jnp.float32),
            pltpu.VMEM_SHARED((HWORDS,), jnp.float32),
            pltpu.VMEM((CHUNK,), jnp.int32),
            pltpu.VMEM((CHUNK,), jnp.int32),
            pltpu.VMEM((CHUNK,), jnp.int32),
            pltpu.VMEM((CHUNK,), jnp.int32),
            pltpu.VMEM((CHUNK,), jnp.int32),
            pltpu.VMEM((CHUNK,), jnp.int32),
            pltpu.VMEM((CHUNK,), jnp.float32),
            pltpu.VMEM((CHUNK, HID), jnp.float32),
            pltpu.SemaphoreType.DMA,
        ],
    )
    return kern(h, srcp, dstp, ttp, etp, zrows, zflat)


def _final_body(a0_ref, a1_ref, g0_ref, g1_ref, h_ref, tab_ref, wo_ref,
                bo_ref, out_ref):
    hist = jnp.concatenate([g0_ref[...], g1_ref[...]], axis=1)
    agg = a0_ref[...] + a1_ref[...]
    agg = agg + jnp.dot(hist, tab_ref[...], preferred_element_type=jnp.float32)
    col = lax.broadcasted_iota(jnp.int32, hist.shape, 1)
    rel_mask = (col >= NTIME) & (col < NTIME + NREL)
    deg = jnp.sum(jnp.where(rel_mask, hist, 0.0), axis=1, keepdims=True)
    agg = agg / jnp.maximum(deg, 1.0)
    z = jnp.dot(agg, wo_ref[...], preferred_element_type=jnp.float32)
    out_ref[...] = jax.nn.gelu(z + bo_ref[...]) + h_ref[...]


def _final(a0, a1, g0, g1, h, table, W_out, b_out):
    blk = 1000
    grid = N // blk
    return pl.pallas_call(
        _final_body,
        grid=(grid,),
        in_specs=[
            pl.BlockSpec((blk, HID), lambda i: (i, 0)),
            pl.BlockSpec((blk, HID), lambda i: (i, 0)),
            pl.BlockSpec((blk, HCOLS), lambda i: (i, 0)),
            pl.BlockSpec((blk, HCOLS), lambda i: (i, 0)),
            pl.BlockSpec((blk, HID), lambda i: (i, 0)),
            pl.BlockSpec((HID, HID), lambda i: (0, 0)),
            pl.BlockSpec((HID, HID), lambda i: (0, 0)),
            pl.BlockSpec((1, HID), lambda i: (0, 0)),
        ],
        out_specs=pl.BlockSpec((blk, HID), lambda i: (i, 0)),
        out_shape=jax.ShapeDtypeStruct((N, HID), jnp.float32),
    )(a0, a1, g0, g1, h, table, W_out, b_out)


def kernel(node_feature, node_type, edge_time, edge_index, edge_type,
           W_in, rel_emb, time_emb, W_out, b_out):
    h = _adapt(node_feature, node_type.astype(jnp.int32), W_in)

    npad = EPAD - E
    src = jnp.concatenate([edge_index[0], jnp.zeros((npad,), jnp.int32)])
    dst = jnp.concatenate([edge_index[1], jnp.full((npad,), N, jnp.int32)])
    ttp = jnp.concatenate([edge_time, jnp.zeros((npad,), jnp.int32)])
    etp = jnp.concatenate([edge_type, jnp.zeros((npad,), jnp.int32)])
    zrows = jnp.zeros((ROWS_PER_TILE, HID), jnp.float32)
    zflat = jnp.zeros((HIST_PER_TILE,), jnp.float32)

    agg_parts, hist_parts = _sc_edge(h, src, dst, ttp, etp, zrows, zflat)

    a0 = agg_parts[0, :N, :]
    a1 = agg_parts[1, :N, :]
    g0 = hist_parts[0].reshape(NPAD, HCOLS)[:N, :]
    g1 = hist_parts[1].reshape(NPAD, HCOLS)[:N, :]

    table = jnp.concatenate(
        [time_emb, rel_emb, jnp.zeros((128 - NTIME - NREL, HID), jnp.float32)],
        axis=0)
    return _final(a0, a1, g0, g1, h, table, W_out, b_out.reshape(1, HID))


# speedup vs baseline: 6.1575x; 1.3892x over previous
import jax
import jax.numpy as jnp
from jax import lax
from jax.experimental import pallas as pl
from jax.experimental.pallas import tpu as pltpu
from jax.experimental.pallas import tpu_sc as plsc

N = 10000
E = 320000
D = 128
HID = 128
NTYPES = 3
NREL = 5
NTIME = 120

NC = 2
NS = 16
G = 16
BATCH = 128
NB = 158
EPAD = NB * BATCH * NS
HSPLIT_B = 79
NPAD = 10112
ROWS_PER_TILE = NPAD // NS
HCOLS = 64
HWORDS = N * HCOLS + 1024
HIST_PER_TILE = HWORDS // NS
TRASH = N * HCOLS


def _adapt_body(x_ref, t_ref, w_ref, h_ref):
    x = x_ref[...]
    h0 = jnp.dot(x, w_ref[0], preferred_element_type=jnp.float32)
    h1 = jnp.dot(x, w_ref[1], preferred_element_type=jnp.float32)
    h2 = jnp.dot(x, w_ref[2], preferred_element_type=jnp.float32)
    t = t_ref[...]
    h_ref[...] = jnp.where(t == 0, h0, jnp.where(t == 1, h1, h2))


def _adapt(node_feature, node_type, W_in):
    blk = 1000
    grid = N // blk
    return pl.pallas_call(
        _adapt_body,
        grid=(grid,),
        in_specs=[
            pl.BlockSpec((blk, D), lambda i: (i, 0)),
            pl.BlockSpec((blk, 1), lambda i: (i, 0)),
            pl.BlockSpec((NTYPES, D, HID), lambda i: (0, 0, 0)),
        ],
        out_specs=pl.BlockSpec((blk, HID), lambda i: (i, 0)),
        out_shape=jax.ShapeDtypeStruct((N, HID), jnp.float32),
    )(node_feature, node_type.reshape(N, 1), W_in)


def _sc_edge_body(h, srcp, megap, zrows, zflat,
                  agg_out, hist_out,
                  agg_sh, hist_sh,
                  src_b, mega_b, f12_b, ones_v,
                  d0, d1, d2, d3, d4, d5, d6, d7,
                  d8, d9, d10, d11, d12, d13, d14, d15,
                  msg_a, msg_b, msg_c,
                  sem_ld, sem_g, sem_s, sem_h):
    c = lax.axis_index("c")
    s = lax.axis_index("s")
    dbufs = (d0, d1, d2, d3, d4, d5, d6, d7,
             d8, d9, d10, d11, d12, d13, d14, d15)

    pltpu.sync_copy(zrows, agg_sh.at[pl.ds(s * ROWS_PER_TILE, ROWS_PER_TILE)])
    pltpu.sync_copy(zflat, hist_sh.at[pl.ds(s * HIST_PER_TILE, HIST_PER_TILE)])
    for k in range(BATCH // 16):
        ones_v[pl.ds(k * 16, 16)] = jnp.full((16,), 1.0, jnp.float32)
    plsc.subcore_barrier()

    lo = c * HCOLS
    tile_e = s * (NB * BATCH)

    def drain512(sem):
        pltpu.make_async_copy(zflat.at[pl.ds(0, 128)], ones_v, sem).wait()

    pltpu.async_copy(srcp.at[pl.ds(tile_e, BATCH)],
                     src_b.at[pl.ds(0, BATCH)], sem_ld)
    pltpu.async_copy(megap.at[pl.ds(tile_e, BATCH)],
                     mega_b.at[pl.ds(0, BATCH)], sem_ld)

    def sub_batch(b, half):
        off = pl.multiple_of(tile_e + b * BATCH, BATCH)

        drain512(sem_ld)
        drain512(sem_ld)

        nhalf = 1 - half

        @pl.when(b + 1 < NB)
        def _():
            noff = pl.multiple_of(off + BATCH, BATCH)
            pltpu.async_copy(srcp.at[pl.ds(noff, BATCH)],
                             src_b.at[pl.ds(nhalf * BATCH, BATCH)], sem_ld)
            pltpu.async_copy(megap.at[pl.ds(noff, BATCH)],
                             mega_b.at[pl.ds(nhalf * BATCH, BATCH)], sem_ld)

        @pl.when(b >= 2)
        def _():
            drain512(sem_h)

        @pl.when((b >= 2) & (c == 1))
        def _():
            drain512(sem_h)

        for g in range(BATCH // 16):
            sl = pl.ds(g * 16, 16)
            m = mega_b[pl.ds(half * BATCH + g * 16, 16)]
            d = m >> 10
            low = m & 1023
            t = low & 127
            e = low >> 7
            d64 = d * HCOLS
            own = (t >= lo) & (t < lo + HCOLS)
            f12_b[half * 2, sl] = jnp.where(own, d64 + (t - lo),
                                            TRASH + ((d64 + t) & 1023))
            f12_b[half * 2 + 1, sl] = d64 + ((NTIME - HCOLS) + e)
            dbufs[half * 8 + g][...] = d

        pltpu.async_copy(ones_v, hist_sh.at[f12_b.at[half * 2]], sem_h,
                         add=True)

        @pl.when(c == 1)
        def _():
            pltpu.async_copy(ones_v, hist_sh.at[f12_b.at[half * 2 + 1]],
                             sem_h, add=True)

        my = jnp.where(c == 0, b < HSPLIT_B, b >= HSPLIT_B)

        @pl.when(my)
        def _():
            hb = half * BATCH
            msgs = (msg_a, msg_b, msg_c)
            nq = BATCH // G
            gs = [None] * nq
            ss = [None] * nq
            for q in range(3):
                gs[q] = pltpu.async_copy(
                    h.at[src_b.at[pl.ds(hb + q * G, G)]], msgs[q], sem_g)
            for q in range(nq):
                if q >= 3:
                    ss[q - 3].wait()
                    gs[q] = pltpu.async_copy(
                        h.at[src_b.at[pl.ds(hb + q * G, G)]], msgs[q % 3],
                        sem_g)
                gs[q].wait()
                ss[q] = pltpu.async_copy(msgs[q % 3],
                                         agg_sh.at[dbufs[half * 8 + q]],
                                         sem_s, add=True)
            for q in range(nq - 3, nq):
                ss[q].wait()

    def body(i, carry):
        sub_batch(2 * i, 0)
        sub_batch(2 * i + 1, 1)
        return carry

    lax.fori_loop(0, NB // 2, body, 0)

    drain512(sem_h)
    drain512(sem_h)

    @pl.when(c == 1)
    def _():
        drain512(sem_h)
        drain512(sem_h)

    plsc.subcore_barrier()

    pltpu.sync_copy(agg_sh.at[pl.ds(s * ROWS_PER_TILE, ROWS_PER_TILE)],
                    agg_out.at[c, pl.ds(s * ROWS_PER_TILE, ROWS_PER_TILE)])
    pltpu.sync_copy(hist_sh.at[pl.ds(s * HIST_PER_TILE, HIST_PER_TILE)],
                    hist_out.at[c, pl.ds(s * HIST_PER_TILE, HIST_PER_TILE)])


def _sc_edge(h, srcp, megap, zrows, zflat):
    mesh = plsc.VectorSubcoreMesh(core_axis_name="c", subcore_axis_name="s",
                                  num_cores=NC, num_subcores=NS)
    kern = pl.kernel(
        _sc_edge_body,
        out_type=[
            jax.ShapeDtypeStruct((NC, NPAD, HID), jnp.float32),
            jax.ShapeDtypeStruct((NC, HWORDS), jnp.float32),
        ],
        mesh=mesh,
        scratch_types=[
            pltpu.VMEM_SHARED((NPAD, HID), jnp.float32),
            pltpu.VMEM_SHARED((HWORDS,), jnp.float32),
            pltpu.VMEM((2 * BATCH,), jnp.int32),
            pltpu.VMEM((2 * BATCH,), jnp.int32),
            pltpu.VMEM((4, BATCH), jnp.int32),
            pltpu.VMEM((BATCH,), jnp.float32),
        ] + [pltpu.VMEM((G,), jnp.int32)] * 16 + [
            pltpu.VMEM((G, HID), jnp.float32),
            pltpu.VMEM((G, HID), jnp.float32),
            pltpu.VMEM((G, HID), jnp.float32),
            pltpu.SemaphoreType.DMA,
            pltpu.SemaphoreType.DMA,
            pltpu.SemaphoreType.DMA,
            pltpu.SemaphoreType.DMA,
        ],
    )
    return kern(h, srcp, megap, zrows, zflat)


def _final_body(a0_ref, a1_ref, g0_ref, g1_ref, h_ref, tab_ref, wo_ref,
                bo_ref, out_ref):
    hist = jnp.concatenate([g0_ref[...], g1_ref[...]], axis=1)
    agg = a0_ref[...] + a1_ref[...]
    agg = agg + jnp.dot(hist, tab_ref[...], preferred_element_type=jnp.float32)
    col = lax.broadcasted_iota(jnp.int32, hist.shape, 1)
    rel_mask = (col >= NTIME) & (col < NTIME + NREL)
    deg = jnp.sum(jnp.where(rel_mask, hist, 0.0), axis=1, keepdims=True)
    agg = agg / jnp.maximum(deg, 1.0)
    z = jnp.dot(agg, wo_ref[...], preferred_element_type=jnp.float32)
    out_ref[...] = jax.nn.gelu(z + bo_ref[...]) + h_ref[...]


def _final(a0, a1, g0, g1, h, table, W_out, b_out):
    blk = 1000
    grid = N // blk
    return pl.pallas_call(
        _final_body,
        grid=(grid,),
        in_specs=[
            pl.BlockSpec((blk, HID), lambda i: (i, 0)),
            pl.BlockSpec((blk, HID), lambda i: (i, 0)),
            pl.BlockSpec((blk, HCOLS), lambda i: (i, 0)),
            pl.BlockSpec((blk, HCOLS), lambda i: (i, 0)),
            pl.BlockSpec((blk, HID), lambda i: (i, 0)),
            pl.BlockSpec((HID, HID), lambda i: (0, 0)),
            pl.BlockSpec((HID, HID), lambda i: (0, 0)),
            pl.BlockSpec((1, HID), lambda i: (0, 0)),
        ],
        out_specs=pl.BlockSpec((blk, HID), lambda i: (i, 0)),
        out_shape=jax.ShapeDtypeStruct((N, HID), jnp.float32),
    )(a0, a1, g0, g1, h, table, W_out, b_out)


def kernel(node_feature, node_type, edge_time, edge_index, edge_type,
           W_in, rel_emb, time_emb, W_out, b_out):
    h = _adapt(node_feature, node_type.astype(jnp.int32), W_in)

    npad = EPAD - E
    src = jnp.concatenate([edge_index[0], jnp.zeros((npad,), jnp.int32)])
    mega = edge_index[1] * 1024 + edge_type * 128 + edge_time
    megap = jnp.concatenate([mega, jnp.full((npad,), N * 1024, jnp.int32)])
    zrows = jnp.zeros((ROWS_PER_TILE, HID), jnp.float32)
    zflat = jnp.zeros((HIST_PER_TILE,), jnp.float32)

    agg_parts, hist_parts = _sc_edge(h, src, megap, zrows, zflat)

    a0 = agg_parts[0, :N, :]
    a1 = agg_parts[1, :N, :]
    g0 = hist_parts[0, :TRASH].reshape(N, HCOLS)
    g1 = hist_parts[1, :TRASH].reshape(N, HCOLS)

    table = jnp.concatenate(
        [time_emb, rel_emb, jnp.zeros((128 - NTIME - NREL, HID), jnp.float32)],
        axis=0)
    return _final(a0, a1, g0, g1, h, table, W_out, b_out.reshape(1, HID))


# speedup vs baseline: 7.3957x; 1.2011x over previous
import jax
import jax.numpy as jnp
from jax import lax
from jax.experimental import pallas as pl
from jax.experimental.pallas import tpu as pltpu
from jax.experimental.pallas import tpu_sc as plsc

N = 10000
E = 320000
D = 128
HID = 128
NTYPES = 3
NREL = 5
NTIME = 120

NC = 2
NS = 16
G = 32
BATCH = 128
NB = 158
EPAD = NB * BATCH * NS
HSPLIT_B = 79
NPAD = 10112
ROWS_PER_TILE = NPAD // NS
HCOLS = 64
HWORDS = N * HCOLS + 1024
HIST_PER_TILE = HWORDS // NS
TRASH = N * HCOLS


def _adapt_body(x_ref, t_ref, w_ref, h_ref):
    x = x_ref[...]
    h0 = jnp.dot(x, w_ref[0], preferred_element_type=jnp.float32)
    h1 = jnp.dot(x, w_ref[1], preferred_element_type=jnp.float32)
    h2 = jnp.dot(x, w_ref[2], preferred_element_type=jnp.float32)
    t = t_ref[...]
    h_ref[...] = jnp.where(t == 0, h0, jnp.where(t == 1, h1, h2))


def _adapt(node_feature, node_type, W_in):
    blk = 1000
    grid = N // blk
    return pl.pallas_call(
        _adapt_body,
        grid=(grid,),
        in_specs=[
            pl.BlockSpec((blk, D), lambda i: (i, 0)),
            pl.BlockSpec((blk, 1), lambda i: (i, 0)),
            pl.BlockSpec((NTYPES, D, HID), lambda i: (0, 0, 0)),
        ],
        out_specs=pl.BlockSpec((blk, HID), lambda i: (i, 0)),
        out_shape=jax.ShapeDtypeStruct((N, HID), jnp.float32),
    )(node_feature, node_type.reshape(N, 1), W_in)


def _sc_edge_body(h, srcp, megap, zrows, zflat,
                  agg_out, hist_out,
                  agg_sh, hist_sh,
                  src_b, mega_b, f12_b, ones_v,
                  d0, d1, d2, d3,
                  msg_a, msg_b,
                  sem_ld, sem_g, sem_s, sem_h):
    c = lax.axis_index("c")
    s = lax.axis_index("s")
    dbufs = (d0, d1, d2, d3)

    pltpu.sync_copy(zrows, agg_sh.at[pl.ds(s * ROWS_PER_TILE, ROWS_PER_TILE)])
    pltpu.sync_copy(zflat, hist_sh.at[pl.ds(s * HIST_PER_TILE, HIST_PER_TILE)])
    for k in range(BATCH // 16):
        ones_v[pl.ds(k * 16, 16)] = jnp.full((16,), 1.0, jnp.float32)
    plsc.subcore_barrier()

    lo = c * HCOLS
    tile_e = s * (NB * BATCH)

    def drain512(sem):
        pltpu.make_async_copy(zflat.at[pl.ds(0, 128)], ones_v, sem).wait()

    pltpu.async_copy(srcp.at[pl.ds(tile_e, BATCH)],
                     src_b.at[pl.ds(0, BATCH)], sem_ld)
    pltpu.async_copy(megap.at[pl.ds(tile_e, BATCH)],
                     mega_b.at[pl.ds(0, BATCH)], sem_ld)

    def sub_batch(b, half):
        off = pl.multiple_of(tile_e + b * BATCH, BATCH)

        drain512(sem_ld)
        drain512(sem_ld)

        nhalf = 1 - half

        @pl.when(b + 1 < NB)
        def _():
            noff = pl.multiple_of(off + BATCH, BATCH)
            pltpu.async_copy(srcp.at[pl.ds(noff, BATCH)],
                             src_b.at[pl.ds(nhalf * BATCH, BATCH)], sem_ld)
            pltpu.async_copy(megap.at[pl.ds(noff, BATCH)],
                             mega_b.at[pl.ds(nhalf * BATCH, BATCH)], sem_ld)

        @pl.when(b >= 2)
        def _():
            drain512(sem_h)

        @pl.when((b >= 2) & (c == 1))
        def _():
            drain512(sem_h)

        for g in range(BATCH // 16):
            sl = pl.ds(g * 16, 16)
            m = mega_b[pl.ds(half * BATCH + g * 16, 16)]
            d = m >> 10
            low = m & 1023
            t = low & 127
            e = low >> 7
            d64 = d * HCOLS
            own = (t >= lo) & (t < lo + HCOLS)
            f12_b[half * 2, sl] = jnp.where(own, d64 + (t - lo),
                                            TRASH + ((d64 + t) & 1023))
            f12_b[half * 2 + 1, sl] = d64 + ((NTIME - HCOLS) + e)
            dbufs[g // 2][pl.ds((g % 2) * 16, 16)] = d

        pltpu.async_copy(ones_v, hist_sh.at[f12_b.at[half * 2]], sem_h,
                         add=True)

        @pl.when(c == 1)
        def _():
            pltpu.async_copy(ones_v, hist_sh.at[f12_b.at[half * 2 + 1]],
                             sem_h, add=True)

        my = jnp.where(c == 0, b < HSPLIT_B, b >= HSPLIT_B)

        @pl.when(my)
        def _():
            hb = half * BATCH
            msgs = (msg_a, msg_b)
            nq = BATCH // G
            gs = [None] * nq
            ss = [None] * nq
            for q in range(2):
                gs[q] = pltpu.async_copy(
                    h.at[src_b.at[pl.ds(hb + q * G, G)]], msgs[q], sem_g)
            for q in range(nq):
                if q >= 2:
                    ss[q - 2].wait()
                    gs[q] = pltpu.async_copy(
                        h.at[src_b.at[pl.ds(hb + q * G, G)]], msgs[q % 2],
                        sem_g)
                gs[q].wait()
                ss[q] = pltpu.async_copy(msgs[q % 2],
                                         agg_sh.at[dbufs[q]],
                                         sem_s, add=True)
            for q in range(nq - 2, nq):
                ss[q].wait()

    def body(i, carry):
        sub_batch(2 * i, 0)
        sub_batch(2 * i + 1, 1)
        return carry

    lax.fori_loop(0, NB // 2, body, 0)

    drain512(sem_h)
    drain512(sem_h)

    @pl.when(c == 1)
    def _():
        drain512(sem_h)
        drain512(sem_h)

    plsc.subcore_barrier()

    pltpu.sync_copy(agg_sh.at[pl.ds(s * ROWS_PER_TILE, ROWS_PER_TILE)],
                    agg_out.at[c, pl.ds(s * ROWS_PER_TILE, ROWS_PER_TILE)])
    pltpu.sync_copy(hist_sh.at[pl.ds(s * HIST_PER_TILE, HIST_PER_TILE)],
                    hist_out.at[c, pl.ds(s * HIST_PER_TILE, HIST_PER_TILE)])


def _sc_edge(h, srcp, megap, zrows, zflat):
    mesh = plsc.VectorSubcoreMesh(core_axis_name="c", subcore_axis_name="s",
                                  num_cores=NC, num_subcores=NS)
    kern = pl.kernel(
        _sc_edge_body,
        out_type=[
            jax.ShapeDtypeStruct((NC, NPAD, HID), jnp.float32),
            jax.ShapeDtypeStruct((NC, HWORDS), jnp.float32),
        ],
        mesh=mesh,
        scratch_types=[
            pltpu.VMEM_SHARED((NPAD, HID), jnp.float32),
            pltpu.VMEM_SHARED((HWORDS,), jnp.float32),
            pltpu.VMEM((2 * BATCH,), jnp.int32),
            pltpu.VMEM((2 * BATCH,), jnp.int32),
            pltpu.VMEM((4, BATCH), jnp.int32),
            pltpu.VMEM((BATCH,), jnp.float32),
        ] + [pltpu.VMEM((G,), jnp.int32)] * 4 + [
            pltpu.VMEM((G, HID), jnp.float32),
            pltpu.VMEM((G, HID), jnp.float32),
            pltpu.SemaphoreType.DMA,
            pltpu.SemaphoreType.DMA,
            pltpu.SemaphoreType.DMA,
            pltpu.SemaphoreType.DMA,
        ],
    )
    return kern(h, srcp, megap, zrows, zflat)


def _final_body(a0_ref, a1_ref, g0_ref, g1_ref, h_ref, tab_ref, wo_ref,
                bo_ref, out_ref):
    hist = jnp.concatenate([g0_ref[...], g1_ref[...]], axis=1)
    agg = a0_ref[...] + a1_ref[...]
    agg = agg + jnp.dot(hist, tab_ref[...], preferred_element_type=jnp.float32)
    col = lax.broadcasted_iota(jnp.int32, hist.shape, 1)
    rel_mask = (col >= NTIME) & (col < NTIME + NREL)
    deg = jnp.sum(jnp.where(rel_mask, hist, 0.0), axis=1, keepdims=True)
    agg = agg / jnp.maximum(deg, 1.0)
    z = jnp.dot(agg, wo_ref[...], preferred_element_type=jnp.float32)
    out_ref[...] = jax.nn.gelu(z + bo_ref[...]) + h_ref[...]


def _final(a0, a1, g0, g1, h, table, W_out, b_out):
    blk = 1000
    grid = N // blk
    return pl.pallas_call(
        _final_body,
        grid=(grid,),
        in_specs=[
            pl.BlockSpec((blk, HID), lambda i: (i, 0)),
            pl.BlockSpec((blk, HID), lambda i: (i, 0)),
            pl.BlockSpec((blk, HCOLS), lambda i: (i, 0)),
            pl.BlockSpec((blk, HCOLS), lambda i: (i, 0)),
            pl.BlockSpec((blk, HID), lambda i: (i, 0)),
            pl.BlockSpec((HID, HID), lambda i: (0, 0)),
            pl.BlockSpec((HID, HID), lambda i: (0, 0)),
            pl.BlockSpec((1, HID), lambda i: (0, 0)),
        ],
        out_specs=pl.BlockSpec((blk, HID), lambda i: (i, 0)),
        out_shape=jax.ShapeDtypeStruct((N, HID), jnp.float32),
    )(a0, a1, g0, g1, h, table, W_out, b_out)


def kernel(node_feature, node_type, edge_time, edge_index, edge_type,
           W_in, rel_emb, time_emb, W_out, b_out):
    h = _adapt(node_feature, node_type.astype(jnp.int32), W_in)

    npad = EPAD - E
    src = jnp.concatenate([edge_index[0], jnp.zeros((npad,), jnp.int32)])
    mega = edge_index[1] * 1024 + edge_type * 128 + edge_time
    megap = jnp.concatenate([mega, jnp.full((npad,), N * 1024, jnp.int32)])
    zrows = jnp.zeros((ROWS_PER_TILE, HID), jnp.float32)
    zflat = jnp.zeros((HIST_PER_TILE,), jnp.float32)

    agg_parts, hist_parts = _sc_edge(h, src, megap, zrows, zflat)

    a0 = agg_parts[0, :N, :]
    a1 = agg_parts[1, :N, :]
    g0 = hist_parts[0, :TRASH].reshape(N, HCOLS)
    g1 = hist_parts[1, :TRASH].reshape(N, HCOLS)

    table = jnp.concatenate(
        [time_emb, rel_emb, jnp.zeros((128 - NTIME - NREL, HID), jnp.float32)],
        axis=0)
    return _final(a0, a1, g0, g1, h, table, W_out, b_out.reshape(1, HID))


# speedup vs baseline: 8.0487x; 1.0883x over previous
import jax
import jax.numpy as jnp
from jax import lax
from jax.experimental import pallas as pl
from jax.experimental.pallas import tpu as pltpu
from jax.experimental.pallas import tpu_sc as plsc

N = 10000
E = 320000
D = 128
HID = 128
NTYPES = 3
NREL = 5
NTIME = 120

NC = 2
NS = 16
G = 32
BATCH = 128
NB = 158
EPAD = NB * BATCH * NS
HSPLIT_B = 99
NPAD = 10112
ROWS_PER_TILE = NPAD // NS
HCOLS = 64
HWORDS = N * HCOLS + 1024
HIST_PER_TILE = HWORDS // NS
TRASH = N * HCOLS


def _adapt_body(x_ref, t_ref, w_ref, h_ref):
    x = x_ref[...]
    h0 = jnp.dot(x, w_ref[0], preferred_element_type=jnp.float32)
    h1 = jnp.dot(x, w_ref[1], preferred_element_type=jnp.float32)
    h2 = jnp.dot(x, w_ref[2], preferred_element_type=jnp.float32)
    t = t_ref[...]
    h_ref[...] = jnp.where(t == 0, h0, jnp.where(t == 1, h1, h2))


def _adapt(node_feature, node_type, W_in):
    blk = 1000
    grid = N // blk
    return pl.pallas_call(
        _adapt_body,
        grid=(grid,),
        in_specs=[
            pl.BlockSpec((blk, D), lambda i: (i, 0)),
            pl.BlockSpec((blk, 1), lambda i: (i, 0)),
            pl.BlockSpec((NTYPES, D, HID), lambda i: (0, 0, 0)),
        ],
        out_specs=pl.BlockSpec((blk, HID), lambda i: (i, 0)),
        out_shape=jax.ShapeDtypeStruct((N, HID), jnp.float32),
    )(node_feature, node_type.reshape(N, 1), W_in)


def _sc_edge_body(h, srcp, megap, zrows, zflat,
                  agg_out, hist_out,
                  agg_sh, hist_sh,
                  src_b, mega_b, f12_b, ones_v,
                  d0, d1, d2, d3,
                  msg_a, msg_b,
                  sem_ld, sem_g, sem_s, sem_h):
    c = lax.axis_index("c")
    s = lax.axis_index("s")
    dbufs = (d0, d1, d2, d3)

    pltpu.sync_copy(zrows, agg_sh.at[pl.ds(s * ROWS_PER_TILE, ROWS_PER_TILE)])
    pltpu.sync_copy(zflat, hist_sh.at[pl.ds(s * HIST_PER_TILE, HIST_PER_TILE)])
    for k in range(BATCH // 16):
        ones_v[pl.ds(k * 16, 16)] = jnp.full((16,), 1.0, jnp.float32)
    plsc.subcore_barrier()

    lo = c * HCOLS
    tile_e = s * (NB * BATCH)

    def drain512(sem):
        pltpu.make_async_copy(zflat.at[pl.ds(0, 128)], ones_v, sem).wait()

    pltpu.async_copy(srcp.at[pl.ds(tile_e, BATCH)],
                     src_b.at[pl.ds(0, BATCH)], sem_ld)
    pltpu.async_copy(megap.at[pl.ds(tile_e, BATCH)],
                     mega_b.at[pl.ds(0, BATCH)], sem_ld)

    def sub_batch(b, half):
        off = pl.multiple_of(tile_e + b * BATCH, BATCH)

        drain512(sem_ld)
        drain512(sem_ld)

        nhalf = 1 - half

        @pl.when(b + 1 < NB)
        def _():
            noff = pl.multiple_of(off + BATCH, BATCH)
            pltpu.async_copy(srcp.at[pl.ds(noff, BATCH)],
                             src_b.at[pl.ds(nhalf * BATCH, BATCH)], sem_ld)
            pltpu.async_copy(megap.at[pl.ds(noff, BATCH)],
                             mega_b.at[pl.ds(nhalf * BATCH, BATCH)], sem_ld)

        @pl.when(b >= 2)
        def _():
            drain512(sem_h)

        @pl.when((b >= 2) & (c == 1))
        def _():
            drain512(sem_h)

        for g in range(BATCH // 16):
            sl = pl.ds(g * 16, 16)
            m = mega_b[pl.ds(half * BATCH + g * 16, 16)]
            d = m >> 10
            low = m & 1023
            t = low & 127
            e = low >> 7
            d64 = d * HCOLS
            own = (t >= lo) & (t < lo + HCOLS)
            f12_b[half * 2, sl] = jnp.where(own, d64 + (t - lo),
                                            TRASH + ((d64 + t) & 1023))
            f12_b[half * 2 + 1, sl] = d64 + ((NTIME - HCOLS) + e)
            dbufs[g // 2][pl.ds((g % 2) * 16, 16)] = d

        pltpu.async_copy(ones_v, hist_sh.at[f12_b.at[half * 2]], sem_h,
                         add=True)

        @pl.when(c == 1)
        def _():
            pltpu.async_copy(ones_v, hist_sh.at[f12_b.at[half * 2 + 1]],
                             sem_h, add=True)

        my = jnp.where(c == 0, b < HSPLIT_B, b >= HSPLIT_B)

        @pl.when(my)
        def _():
            hb = half * BATCH
            msgs = (msg_a, msg_b)
            nq = BATCH // G
            gs = [None] * nq
            ss = [None] * nq
            for q in range(2):
                gs[q] = pltpu.async_copy(
                    h.at[src_b.at[pl.ds(hb + q * G, G)]], msgs[q], sem_g)
            for q in range(nq):
                if q >= 2:
                    ss[q - 2].wait()
                    gs[q] = pltpu.async_copy(
                        h.at[src_b.at[pl.ds(hb + q * G, G)]], msgs[q % 2],
                        sem_g)
                gs[q].wait()
                ss[q] = pltpu.async_copy(msgs[q % 2],
                                         agg_sh.at[dbufs[q]],
                                         sem_s, add=True)
            for q in range(nq - 2, nq):
                ss[q].wait()

    def body(i, carry):
        sub_batch(2 * i, 0)
        sub_batch(2 * i + 1, 1)
        return carry

    lax.fori_loop(0, NB // 2, body, 0)

    drain512(sem_h)
    drain512(sem_h)

    @pl.when(c == 1)
    def _():
        drain512(sem_h)
        drain512(sem_h)

    plsc.subcore_barrier()

    pltpu.sync_copy(agg_sh.at[pl.ds(s * ROWS_PER_TILE, ROWS_PER_TILE)],
                    agg_out.at[c, pl.ds(s * ROWS_PER_TILE, ROWS_PER_TILE)])
    pltpu.sync_copy(hist_sh.at[pl.ds(s * HIST_PER_TILE, HIST_PER_TILE)],
                    hist_out.at[c, pl.ds(s * HIST_PER_TILE, HIST_PER_TILE)])


def _sc_edge(h, srcp, megap, zrows, zflat):
    mesh = plsc.VectorSubcoreMesh(core_axis_name="c", subcore_axis_name="s",
                                  num_cores=NC, num_subcores=NS)
    kern = pl.kernel(
        _sc_edge_body,
        out_type=[
            jax.ShapeDtypeStruct((NC, NPAD, HID), jnp.float32),
            jax.ShapeDtypeStruct((NC, HWORDS), jnp.float32),
        ],
        mesh=mesh,
        scratch_types=[
            pltpu.VMEM_SHARED((NPAD, HID), jnp.float32),
            pltpu.VMEM_SHARED((HWORDS,), jnp.float32),
            pltpu.VMEM((2 * BATCH,), jnp.int32),
            pltpu.VMEM((2 * BATCH,), jnp.int32),
            pltpu.VMEM((4, BATCH), jnp.int32),
            pltpu.VMEM((BATCH,), jnp.float32),
        ] + [pltpu.VMEM((G,), jnp.int32)] * 4 + [
            pltpu.VMEM((G, HID), jnp.float32),
            pltpu.VMEM((G, HID), jnp.float32),
            pltpu.SemaphoreType.DMA,
            pltpu.SemaphoreType.DMA,
            pltpu.SemaphoreType.DMA,
            pltpu.SemaphoreType.DMA,
        ],
    )
    return kern(h, srcp, megap, zrows, zflat)


def _final_body(a0_ref, a1_ref, g0_ref, g1_ref, h_ref, tab_ref, wo_ref,
                bo_ref, out_ref):
    hist = jnp.concatenate([g0_ref[...], g1_ref[...]], axis=1)
    agg = a0_ref[...] + a1_ref[...]
    agg = agg + jnp.dot(hist, tab_ref[...], preferred_element_type=jnp.float32)
    col = lax.broadcasted_iota(jnp.int32, hist.shape, 1)
    rel_mask = (col >= NTIME) & (col < NTIME + NREL)
    deg = jnp.sum(jnp.where(rel_mask, hist, 0.0), axis=1, keepdims=True)
    agg = agg / jnp.maximum(deg, 1.0)
    z = jnp.dot(agg, wo_ref[...], preferred_element_type=jnp.float32)
    out_ref[...] = jax.nn.gelu(z + bo_ref[...]) + h_ref[...]


def _final(a0, a1, g0, g1, h, table, W_out, b_out):
    blk = 1000
    grid = N // blk
    return pl.pallas_call(
        _final_body,
        grid=(grid,),
        in_specs=[
            pl.BlockSpec((blk, HID), lambda i: (i, 0)),
            pl.BlockSpec((blk, HID), lambda i: (i, 0)),
            pl.BlockSpec((blk, HCOLS), lambda i: (i, 0)),
            pl.BlockSpec((blk, HCOLS), lambda i: (i, 0)),
            pl.BlockSpec((blk, HID), lambda i: (i, 0)),
            pl.BlockSpec((HID, HID), lambda i: (0, 0)),
            pl.BlockSpec((HID, HID), lambda i: (0, 0)),
            pl.BlockSpec((1, HID), lambda i: (0, 0)),
        ],
        out_specs=pl.BlockSpec((blk, HID), lambda i: (i, 0)),
        out_shape=jax.ShapeDtypeStruct((N, HID), jnp.float32),
    )(a0, a1, g0, g1, h, table, W_out, b_out)


def kernel(node_feature, node_type, edge_time, edge_index, edge_type,
           W_in, rel_emb, time_emb, W_out, b_out):
    h = _adapt(node_feature, node_type.astype(jnp.int32), W_in)

    npad = EPAD - E
    src = jnp.concatenate([edge_index[0], jnp.zeros((npad,), jnp.int32)])
    mega = edge_index[1] * 1024 + edge_type * 128 + edge_time
    megap = jnp.concatenate([mega, jnp.full((npad,), N * 1024, jnp.int32)])
    zrows = jnp.zeros((ROWS_PER_TILE, HID), jnp.float32)
    zflat = jnp.zeros((HIST_PER_TILE,), jnp.float32)

    agg_parts, hist_parts = _sc_edge(h, src, megap, zrows, zflat)

    a0 = agg_parts[0, :N, :]
    a1 = agg_parts[1, :N, :]
    g0 = hist_parts[0, :TRASH].reshape(N, HCOLS)
    g1 = hist_parts[1, :TRASH].reshape(N, HCOLS)

    table = jnp.concatenate(
        [time_emb, rel_emb, jnp.zeros((128 - NTIME - NREL, HID), jnp.float32)],
        axis=0)
    return _final(a0, a1, g0, g1, h, table, W_out, b_out.reshape(1, HID))


# speedup vs baseline: 8.7037x; 1.0814x over previous
import jax
import jax.numpy as jnp
from jax import lax
from jax.experimental import pallas as pl
from jax.experimental.pallas import tpu as pltpu
from jax.experimental.pallas import tpu_sc as plsc

N = 10000
E = 320000
D = 128
HID = 128
NTYPES = 3
NREL = 5
NTIME = 120

NC = 2
NS = 16
G = 32
BATCH = 128
NB = 158
EPAD = NB * BATCH * NS
HSPLIT_B = 101
NPAD = 10112
ROWS_PER_TILE = NPAD // NS
HCOLS = 64
HWORDS = N * HCOLS + 1024
HIST_PER_TILE = HWORDS // NS
TRASH = N * HCOLS


def _adapt_body(x_ref, t_ref, w_ref, h_ref):
    x = x_ref[...]
    h0 = jnp.dot(x, w_ref[0], preferred_element_type=jnp.float32)
    h1 = jnp.dot(x, w_ref[1], preferred_element_type=jnp.float32)
    h2 = jnp.dot(x, w_ref[2], preferred_element_type=jnp.float32)
    t = t_ref[...]
    h_ref[...] = jnp.where(t == 0, h0, jnp.where(t == 1, h1, h2))


def _adapt(node_feature, node_type, W_in):
    blk = 1000
    grid = N // blk
    return pl.pallas_call(
        _adapt_body,
        grid=(grid,),
        in_specs=[
            pl.BlockSpec((blk, D), lambda i: (i, 0)),
            pl.BlockSpec((blk, 1), lambda i: (i, 0)),
            pl.BlockSpec((NTYPES, D, HID), lambda i: (0, 0, 0)),
        ],
        out_specs=pl.BlockSpec((blk, HID), lambda i: (i, 0)),
        out_shape=jax.ShapeDtypeStruct((N, HID), jnp.float32),
    )(node_feature, node_type.reshape(N, 1), W_in)


def _sc_edge_body(h, srcp, megap, zrows, zflat,
                  agg_out, hist_out,
                  agg_sh, hist_sh,
                  src_b, mega_b, f12_b, ones_v,
                  d0, d1, d2, d3,
                  msg_a, msg_b,
                  sem_ld, sem_g, sem_s, sem_h):
    c = lax.axis_index("c")
    s = lax.axis_index("s")
    dbufs = (d0, d1, d2, d3)

    pltpu.sync_copy(zrows, agg_sh.at[pl.ds(s * ROWS_PER_TILE, ROWS_PER_TILE)])
    pltpu.sync_copy(zflat, hist_sh.at[pl.ds(s * HIST_PER_TILE, HIST_PER_TILE)])
    for k in range(BATCH // 16):
        ones_v[pl.ds(k * 16, 16)] = jnp.full((16,), 1.0, jnp.float32)
    plsc.subcore_barrier()

    lo = c * HCOLS
    tile_e = s * (NB * BATCH)

    def drain512(sem):
        pltpu.make_async_copy(zflat.at[pl.ds(0, 128)], ones_v, sem).wait()

    pltpu.async_copy(srcp.at[pl.ds(tile_e, BATCH)],
                     src_b.at[pl.ds(0, BATCH)], sem_ld)
    pltpu.async_copy(megap.at[pl.ds(tile_e, BATCH)],
                     mega_b.at[pl.ds(0, BATCH)], sem_ld)

    def sub_batch(b, half):
        off = pl.multiple_of(tile_e + b * BATCH, BATCH)

        drain512(sem_ld)
        drain512(sem_ld)

        nhalf = 1 - half

        @pl.when(b + 1 < NB)
        def _():
            noff = pl.multiple_of(off + BATCH, BATCH)
            pltpu.async_copy(srcp.at[pl.ds(noff, BATCH)],
                             src_b.at[pl.ds(nhalf * BATCH, BATCH)], sem_ld)
            pltpu.async_copy(megap.at[pl.ds(noff, BATCH)],
                             mega_b.at[pl.ds(nhalf * BATCH, BATCH)], sem_ld)

        @pl.when(b >= 2)
        def _():
            drain512(sem_h)

        @pl.when((b >= 2) & (c == 1))
        def _():
            drain512(sem_h)

        for g in range(BATCH // 16):
            sl = pl.ds(g * 16, 16)
            m = mega_b[pl.ds(half * BATCH + g * 16, 16)]
            d = m >> 10
            low = m & 1023
            t = low & 127
            e = low >> 7
            d64 = d * HCOLS
            own = (t >= lo) & (t < lo + HCOLS)
            f12_b[half * 2, sl] = jnp.where(own, d64 + (t - lo),
                                            TRASH + ((d64 + t) & 1023))
            f12_b[half * 2 + 1, sl] = d64 + ((NTIME - HCOLS) + e)
            dbufs[g // 2][pl.ds((g % 2) * 16, 16)] = d

        pltpu.async_copy(ones_v, hist_sh.at[f12_b.at[half * 2]], sem_h,
                         add=True)

        @pl.when(c == 1)
        def _():
            pltpu.async_copy(ones_v, hist_sh.at[f12_b.at[half * 2 + 1]],
                             sem_h, add=True)

        my = jnp.where(c == 0, b < HSPLIT_B, b >= HSPLIT_B)

        @pl.when(my)
        def _():
            hb = half * BATCH
            msgs = (msg_a, msg_b)
            nq = BATCH // G
            gs = [None] * nq
            ss = [None] * nq
            for q in range(2):
                gs[q] = pltpu.async_copy(
                    h.at[src_b.at[pl.ds(hb + q * G, G)]], msgs[q], sem_g)
            for q in range(nq):
                if q >= 2:
                    ss[q - 2].wait()
                    gs[q] = pltpu.async_copy(
                        h.at[src_b.at[pl.ds(hb + q * G, G)]], msgs[q % 2],
                        sem_g)
                gs[q].wait()
                ss[q] = pltpu.async_copy(msgs[q % 2],
                                         agg_sh.at[dbufs[q]],
                                         sem_s, add=True)
            for q in range(nq - 2, nq):
                ss[q].wait()

    def body(i, carry):
        sub_batch(2 * i, 0)
        sub_batch(2 * i + 1, 1)
        return carry

    lax.fori_loop(0, NB // 2, body, 0)

    drain512(sem_h)
    drain512(sem_h)

    @pl.when(c == 1)
    def _():
        drain512(sem_h)
        drain512(sem_h)

    plsc.subcore_barrier()

    pltpu.sync_copy(agg_sh.at[pl.ds(s * ROWS_PER_TILE, ROWS_PER_TILE)],
                    agg_out.at[c, pl.ds(s * ROWS_PER_TILE, ROWS_PER_TILE)])
    pltpu.sync_copy(hist_sh.at[pl.ds(s * HIST_PER_TILE, HIST_PER_TILE)],
                    hist_out.at[c, pl.ds(s * HIST_PER_TILE, HIST_PER_TILE)])


def _sc_edge(h, srcp, megap, zrows, zflat):
    mesh = plsc.VectorSubcoreMesh(core_axis_name="c", subcore_axis_name="s",
                                  num_cores=NC, num_subcores=NS)
    kern = pl.kernel(
        _sc_edge_body,
        out_type=[
            jax.ShapeDtypeStruct((NC, NPAD, HID), jnp.float32),
            jax.ShapeDtypeStruct((NC, HWORDS), jnp.float32),
        ],
        mesh=mesh,
        scratch_types=[
            pltpu.VMEM_SHARED((NPAD, HID), jnp.float32),
            pltpu.VMEM_SHARED((HWORDS,), jnp.float32),
            pltpu.VMEM((2 * BATCH,), jnp.int32),
            pltpu.VMEM((2 * BATCH,), jnp.int32),
            pltpu.VMEM((4, BATCH), jnp.int32),
            pltpu.VMEM((BATCH,), jnp.float32),
        ] + [pltpu.VMEM((G,), jnp.int32)] * 4 + [
            pltpu.VMEM((G, HID), jnp.float32),
            pltpu.VMEM((G, HID), jnp.float32),
            pltpu.SemaphoreType.DMA,
            pltpu.SemaphoreType.DMA,
            pltpu.SemaphoreType.DMA,
            pltpu.SemaphoreType.DMA,
        ],
    )
    return kern(h, srcp, megap, zrows, zflat)


def _final_body(ap_ref, hp_ref, h_ref, tab_ref, wo_ref,
                bo_ref, out_ref):
    hist = jnp.concatenate([hp_ref[0], hp_ref[1]], axis=1)
    agg = ap_ref[0] + ap_ref[1]
    agg = agg + jnp.dot(hist, tab_ref[...], preferred_element_type=jnp.float32)
    col = lax.broadcasted_iota(jnp.int32, hist.shape, 1)
    rel_mask = (col >= NTIME) & (col < NTIME + NREL)
    deg = jnp.sum(jnp.where(rel_mask, hist, 0.0), axis=1, keepdims=True)
    agg = agg / jnp.maximum(deg, 1.0)
    z = jnp.dot(agg, wo_ref[...], preferred_element_type=jnp.float32)
    out_ref[...] = jax.nn.gelu(z + bo_ref[...]) + h_ref[...]


def _final(agg_parts, hist_rows, h, table, W_out, b_out):
    blk = 1000
    grid = N // blk
    return pl.pallas_call(
        _final_body,
        grid=(grid,),
        in_specs=[
            pl.BlockSpec((NC, blk, HID), lambda i: (0, i, 0)),
            pl.BlockSpec((NC, blk, HCOLS), lambda i: (0, i, 0)),
            pl.BlockSpec((blk, HID), lambda i: (i, 0)),
            pl.BlockSpec((HID, HID), lambda i: (0, 0)),
            pl.BlockSpec((HID, HID), lambda i: (0, 0)),
            pl.BlockSpec((1, HID), lambda i: (0, 0)),
        ],
        out_specs=pl.BlockSpec((blk, HID), lambda i: (i, 0)),
        out_shape=jax.ShapeDtypeStruct((N, HID), jnp.float32),
    )(agg_parts, hist_rows, h, table, W_out, b_out)


def kernel(node_feature, node_type, edge_time, edge_index, edge_type,
           W_in, rel_emb, time_emb, W_out, b_out):
    h = _adapt(node_feature, node_type.astype(jnp.int32), W_in)

    npad = EPAD - E
    src = jnp.concatenate([edge_index[0], jnp.zeros((npad,), jnp.int32)])
    mega = edge_index[1] * 1024 + edge_type * 128 + edge_time
    megap = jnp.concatenate([mega, jnp.full((npad,), N * 1024, jnp.int32)])
    zrows = jnp.zeros((ROWS_PER_TILE, HID), jnp.float32)
    zflat = jnp.zeros((HIST_PER_TILE,), jnp.float32)

    agg_parts, hist_parts = _sc_edge(h, src, megap, zrows, zflat)

    hist_rows = hist_parts.reshape(NC, HWORDS // HCOLS, HCOLS)

    table = jnp.concatenate(
        [time_emb, rel_emb, jnp.zeros((128 - NTIME - NREL, HID), jnp.float32)],
        axis=0)
    return _final(agg_parts, hist_rows, h, table, W_out, b_out.reshape(1, HID))
